# baseline (speedup 1.0000x reference)
import functools

import jax
import jax.numpy as jnp
from jax import lax
from jax.experimental import pallas as pl
from jax.experimental.pallas import tpu as pltpu
from jax.experimental.pallas import tpu_sc as plsc

N_FIELDS = 26
VOCAB = 100000
EMBED_DIM = 32
BATCH = 16384

NC, NS, LANES = 2, 16, 16
NW = NC * NS

VBLK = 512
NVBLK = -(-VOCAB // VBLK)
VPAD = NVBLK * VBLK
ROWS128 = VBLK * EMBED_DIM // 128
FLAT_ROWS = N_FIELDS * NVBLK * ROWS128

BPW = BATCH // NW
CB = 4
NCHUNK = BPW // CB
CHUNK_IDX = CB * N_FIELDS
NBUF = 4


def _transpose_body(in_ref, out_ref):
    blk = in_ref[0]
    out_ref[...] = jnp.concatenate(
        [blk[:, 128 * p:128 * (p + 1)].T for p in range(4)], axis=1
    )


_transpose_flat = pl.pallas_call(
    _transpose_body,
    grid=(N_FIELDS, NVBLK),
    in_specs=[
        pl.BlockSpec((1, EMBED_DIM, VBLK), lambda f, v: (f, 0, v)),
    ],
    out_specs=pl.BlockSpec((ROWS128, 128), lambda f, v: (f * NVBLK + v, 0)),
    out_shape=jax.ShapeDtypeStruct((FLAT_ROWS, 128), jnp.float32),
)


def _make_gather_kernel():
    mesh = plsc.VectorSubcoreMesh(core_axis_name="c", subcore_axis_name="s")

    @functools.partial(
        pl.kernel,
        mesh=mesh,
        out_type=jax.ShapeDtypeStruct((BATCH, EMBED_DIM), jnp.float32),
        compiler_params=pltpu.CompilerParams(use_tc_tiling_on_sc=False),
        scratch_types=(
            [pltpu.VMEM((NCHUNK, CHUNK_IDX), jnp.int32),
             pltpu.VMEM((BPW, EMBED_DIM), jnp.float32)]
            + [pltpu.VMEM((CHUNK_IDX, EMBED_DIM), jnp.float32)
               for _ in range(NBUF)]
            + [pltpu.SemaphoreType.DMA for _ in range(NBUF)]
        ),
    )
    def emb_kernel(table_hbm, idx_hbm, out_hbm, idx_v, out_v, *bufs_sems):
        bufs = bufs_sems[:NBUF]
        sems = bufs_sems[NBUF:]
        wid = lax.axis_index("s") * NC + lax.axis_index("c")

        pltpu.sync_copy(idx_hbm.at[wid], idx_v)

        def start(c, b):
            @pl.when(c < NCHUNK)
            def _():
                pltpu.async_copy(table_hbm.at[idx_v.at[c]], bufs[b], sems[b])

        def wait(b):
            pltpu.make_async_copy(
                table_hbm.at[pl.ds(0, CHUNK_IDX)], bufs[b], sems[b]
            ).wait()

        for b in range(NBUF):
            start(b, b)

        def body(g, carry):
            for b in range(NBUF):
                c = g * NBUF + b
                wait(b)
                buf = bufs[b]
                for j in range(CB):
                    r0 = j * N_FIELDS
                    acc0 = buf[r0, pl.ds(0, LANES)]
                    acc1 = buf[r0, pl.ds(LANES, LANES)]
                    for f in range(1, N_FIELDS):
                        acc0 = acc0 + buf[r0 + f, pl.ds(0, LANES)]
                        acc1 = acc1 + buf[r0 + f, pl.ds(LANES, LANES)]
                    out_r = c * CB + j
                    out_v[out_r, pl.ds(0, LANES)] = acc0
                    out_v[out_r, pl.ds(LANES, LANES)] = acc1
                start(c + NBUF, b)
            return carry

        lax.fori_loop(0, NCHUNK // NBUF, body, 0)

        pltpu.sync_copy(out_v, out_hbm.at[pl.ds(wid * BPW, BPW)])

    return emb_kernel


_gather_kernel = _make_gather_kernel()


@jax.jit
def kernel(x, tables):
    tabt = jnp.transpose(tables, (0, 2, 1))
    flat128 = _transpose_flat(tabt)
    flat = flat128.reshape(FLAT_ROWS * 4, EMBED_DIM)
    xi = x.astype(jnp.int32)
    f_base = (jnp.arange(N_FIELDS, dtype=jnp.int32) * (NVBLK * 512))[None, :]
    rows = (
        f_base
        + (xi // VBLK) * 512
        + (xi % 128) * 4
        + (xi // 128) % 4
    )
    flat_idx = rows.reshape(NW, NCHUNK, CHUNK_IDX)
    return _gather_kernel(flat, flat_idx)


# speedup vs baseline: 4.1033x; 4.1033x over previous
import functools

import jax
import jax.numpy as jnp
from jax import lax
from jax.experimental import pallas as pl
from jax.experimental.pallas import tpu as pltpu
from jax.experimental.pallas import tpu_sc as plsc

N_FIELDS = 26
VOCAB = 100000
EMBED_DIM = 32
BATCH = 16384

NC, NS, LANES = 2, 16, 16
NW = NC * NS

VBLK = 4096
NVBLK = -(-VOCAB // VBLK)
VPAD = NVBLK * VBLK
ROWS128 = VBLK * EMBED_DIM // 128
FLAT_ROWS = N_FIELDS * NVBLK * ROWS128

BPW = BATCH // NW
CB = 4
NCHUNK = BPW // CB
CHUNK_IDX = CB * N_FIELDS
NBUF = 4


def _transpose_body(in_ref, out_ref):
    blk = in_ref[0]
    d_io = lax.broadcasted_iota(jnp.int32, (EMBED_DIM, 128), 0)
    c_io = lax.broadcasted_iota(jnp.int32, (EMBED_DIM, 128), 1)
    sel = [
        jnp.where(c_io == d_io + EMBED_DIM * p, 1.0, 0.0)
        for p in range(4)
    ]
    for q in range(VBLK // 512):
        acc = None
        for p in range(4):
            t = lax.dot_general(
                blk[:, 128 * (4 * q + p):128 * (4 * q + p + 1)],
                sel[p],
                (((0,), (0,)), ((), ())),
            )
            acc = t if acc is None else acc + t
        out_ref[q * 128:(q + 1) * 128, :] = acc


_transpose_flat = pl.pallas_call(
    _transpose_body,
    grid=(N_FIELDS, NVBLK),
    in_specs=[
        pl.BlockSpec((1, EMBED_DIM, VBLK), lambda f, v: (f, 0, v)),
    ],
    out_specs=pl.BlockSpec((ROWS128, 128), lambda f, v: (f * NVBLK + v, 0)),
    out_shape=jax.ShapeDtypeStruct((FLAT_ROWS, 128), jnp.float32),
)


def _make_gather_kernel():
    mesh = plsc.VectorSubcoreMesh(core_axis_name="c", subcore_axis_name="s")

    @functools.partial(
        pl.kernel,
        mesh=mesh,
        out_type=jax.ShapeDtypeStruct((BATCH, EMBED_DIM), jnp.float32),
        compiler_params=pltpu.CompilerParams(use_tc_tiling_on_sc=False),
        scratch_types=(
            [pltpu.VMEM((NCHUNK, CHUNK_IDX), jnp.int32),
             pltpu.VMEM((BPW, EMBED_DIM), jnp.float32)]
            + [pltpu.VMEM((CHUNK_IDX, EMBED_DIM), jnp.float32)
               for _ in range(NBUF)]
            + [pltpu.SemaphoreType.DMA for _ in range(NBUF)]
        ),
    )
    def emb_kernel(table_hbm, idx_hbm, out_hbm, idx_v, out_v, *bufs_sems):
        bufs = bufs_sems[:NBUF]
        sems = bufs_sems[NBUF:]
        wid = lax.axis_index("s") * NC + lax.axis_index("c")

        pltpu.sync_copy(idx_hbm.at[wid], idx_v)

        def start(c, b):
            @pl.when(c < NCHUNK)
            def _():
                pltpu.async_copy(table_hbm.at[idx_v.at[c]], bufs[b], sems[b])

        def wait(b):
            pltpu.make_async_copy(
                table_hbm.at[pl.ds(0, CHUNK_IDX)], bufs[b], sems[b]
            ).wait()

        for b in range(NBUF):
            start(b, b)

        def body(g, carry):
            for b in range(NBUF):
                c = g * NBUF + b
                wait(b)
                buf = bufs[b]
                for j in range(CB):
                    r0 = j * N_FIELDS
                    acc0 = buf[r0, pl.ds(0, LANES)]
                    acc1 = buf[r0, pl.ds(LANES, LANES)]
                    for f in range(1, N_FIELDS):
                        acc0 = acc0 + buf[r0 + f, pl.ds(0, LANES)]
                        acc1 = acc1 + buf[r0 + f, pl.ds(LANES, LANES)]
                    out_r = c * CB + j
                    out_v[out_r, pl.ds(0, LANES)] = acc0
                    out_v[out_r, pl.ds(LANES, LANES)] = acc1
                start(c + NBUF, b)
            return carry

        lax.fori_loop(0, NCHUNK // NBUF, body, 0)

        pltpu.sync_copy(out_v, out_hbm.at[pl.ds(wid * BPW, BPW)])

    return emb_kernel


_gather_kernel = _make_gather_kernel()


@jax.jit
def kernel(x, tables):
    tabt = jnp.transpose(tables, (0, 2, 1))
    flat128 = _transpose_flat(tabt)
    flat = flat128.reshape(FLAT_ROWS * 4, EMBED_DIM)
    xi = x.astype(jnp.int32)
    f_base = (jnp.arange(N_FIELDS, dtype=jnp.int32) * (NVBLK * 512))[None, :]
    rows = (
        f_base
        + (xi // VBLK) * 512
        + (xi % 128) * 4
        + (xi // 128) % 4
    )
    flat_idx = rows.reshape(NW, NCHUNK, CHUNK_IDX)
    return _gather_kernel(flat, flat_idx)


# speedup vs baseline: 4.4278x; 1.0791x over previous
import functools

import jax
import jax.numpy as jnp
from jax import lax
from jax.experimental import pallas as pl
from jax.experimental.pallas import tpu as pltpu
from jax.experimental.pallas import tpu_sc as plsc

N_FIELDS = 26
VOCAB = 100000
EMBED_DIM = 32
BATCH = 16384

NC, NS, LANES = 2, 16, 16
NW = NC * NS

VBLK = 12544
NVBLK = -(-VOCAB // VBLK)
VPAD = NVBLK * VBLK
ROWS128 = VBLK * EMBED_DIM // 128
FLAT_ROWS = N_FIELDS * NVBLK * ROWS128

BPW = BATCH // NW
CB = 4
NCHUNK = BPW // CB
CHUNK_IDX = CB * N_FIELDS
NBUF = 4


def _transpose_body(in_ref, out_ref):
    blk = in_ref[0]
    for q in range(VBLK // 512):
        out_ref[q * 128:(q + 1) * 128, :] = jnp.concatenate(
            [
                blk[:, 128 * (4 * q + p):128 * (4 * q + p + 1)].T
                for p in range(4)
            ],
            axis=1,
        )


_transpose_flat = pl.pallas_call(
    _transpose_body,
    grid=(N_FIELDS, NVBLK),
    in_specs=[
        pl.BlockSpec((1, EMBED_DIM, VBLK), lambda f, v: (f, 0, v)),
    ],
    out_specs=pl.BlockSpec((ROWS128, 128), lambda f, v: (f * NVBLK + v, 0)),
    out_shape=jax.ShapeDtypeStruct((FLAT_ROWS, 128), jnp.float32),
)


def _make_gather_kernel():
    mesh = plsc.VectorSubcoreMesh(core_axis_name="c", subcore_axis_name="s")

    @functools.partial(
        pl.kernel,
        mesh=mesh,
        out_type=jax.ShapeDtypeStruct((BATCH, EMBED_DIM), jnp.float32),
        compiler_params=pltpu.CompilerParams(use_tc_tiling_on_sc=False),
        scratch_types=(
            [pltpu.VMEM((NCHUNK, CHUNK_IDX), jnp.int32),
             pltpu.VMEM((BPW, EMBED_DIM), jnp.float32)]
            + [pltpu.VMEM((CHUNK_IDX, EMBED_DIM), jnp.float32)
               for _ in range(NBUF)]
            + [pltpu.SemaphoreType.DMA for _ in range(NBUF)]
        ),
    )
    def emb_kernel(table_hbm, idx_hbm, out_hbm, idx_v, out_v, *bufs_sems):
        bufs = bufs_sems[:NBUF]
        sems = bufs_sems[NBUF:]
        wid = lax.axis_index("s") * NC + lax.axis_index("c")

        pltpu.sync_copy(idx_hbm.at[wid], idx_v)

        def start(c, b):
            @pl.when(c < NCHUNK)
            def _():
                pltpu.async_copy(table_hbm.at[idx_v.at[c]], bufs[b], sems[b])

        def wait(b):
            pltpu.make_async_copy(
                table_hbm.at[pl.ds(0, CHUNK_IDX)], bufs[b], sems[b]
            ).wait()

        for b in range(NBUF):
            start(b, b)

        def body(g, carry):
            for b in range(NBUF):
                c = g * NBUF + b
                wait(b)
                buf = bufs[b]
                for j in range(CB):
                    r0 = j * N_FIELDS
                    acc0 = buf[r0, pl.ds(0, LANES)]
                    acc1 = buf[r0, pl.ds(LANES, LANES)]
                    for f in range(1, N_FIELDS):
                        acc0 = acc0 + buf[r0 + f, pl.ds(0, LANES)]
                        acc1 = acc1 + buf[r0 + f, pl.ds(LANES, LANES)]
                    out_r = c * CB + j
                    out_v[out_r, pl.ds(0, LANES)] = acc0
                    out_v[out_r, pl.ds(LANES, LANES)] = acc1
                start(c + NBUF, b)
            return carry

        lax.fori_loop(0, NCHUNK // NBUF, body, 0)

        pltpu.sync_copy(out_v, out_hbm.at[pl.ds(wid * BPW, BPW)])

    return emb_kernel


_gather_kernel = _make_gather_kernel()


@jax.jit
def kernel(x, tables):
    tabt = jnp.transpose(tables, (0, 2, 1))
    flat128 = _transpose_flat(tabt)
    flat = flat128.reshape(FLAT_ROWS * 4, EMBED_DIM)
    xi = x.astype(jnp.int32)
    f_base = (jnp.arange(N_FIELDS, dtype=jnp.int32) * (NVBLK * 512))[None, :]
    rows = (
        f_base
        + (xi // VBLK) * 512
        + (xi % 128) * 4
        + (xi // 128) % 4
    )
    flat_idx = rows.reshape(NW, NCHUNK, CHUNK_IDX)
    return _gather_kernel(flat, flat_idx)


# speedup vs baseline: 6.1235x; 1.3830x over previous
import functools

import jax
import jax.numpy as jnp
from jax import lax
from jax.experimental import pallas as pl
from jax.experimental.pallas import tpu as pltpu
from jax.experimental.pallas import tpu_sc as plsc

N_FIELDS = 26
VOCAB = 100000
EMBED_DIM = 32
BATCH = 16384

NC, NS, LANES = 2, 16, 16
NW = NC * NS

VBLK = 12800
NVBLK = -(-VOCAB // VBLK)
VPAD = NVBLK * VBLK
ROWS128 = VBLK * EMBED_DIM // 128
FLAT_ROWS = N_FIELDS * NVBLK * ROWS128

BPW = BATCH // NW
CB = 4
NCHUNK = BPW // CB
CHUNK_IDX = CB * N_FIELDS
NBUF = 4


def _transpose_body(in_ref, out_ref):
    blk = in_ref[0]
    d_io = lax.broadcasted_iota(jnp.int32, (EMBED_DIM, 128), 0)
    c_io = lax.broadcasted_iota(jnp.int32, (EMBED_DIM, 128), 1)
    sel = [
        jnp.where(c_io == d_io + EMBED_DIM * p, 1.0, 0.0)
        for p in range(4)
    ]
    for q in range(VBLK // 512):
        acc = None
        for p in range(4):
            t = lax.dot_general(
                blk[:, 128 * (4 * q + p):128 * (4 * q + p + 1)],
                sel[p],
                (((0,), (0,)), ((), ())),
            )
            acc = t if acc is None else acc + t
        out_ref[q * 128:(q + 1) * 128, :] = acc


_transpose_flat = pl.pallas_call(
    _transpose_body,
    grid=(N_FIELDS, NVBLK),
    in_specs=[
        pl.BlockSpec((1, EMBED_DIM, VBLK), lambda f, v: (f, 0, v)),
    ],
    out_specs=pl.BlockSpec((ROWS128, 128), lambda f, v: (f * NVBLK + v, 0)),
    out_shape=jax.ShapeDtypeStruct((FLAT_ROWS, 128), jnp.float32),
)


def _make_gather_kernel():
    mesh = plsc.VectorSubcoreMesh(core_axis_name="c", subcore_axis_name="s")

    @functools.partial(
        pl.kernel,
        mesh=mesh,
        out_type=jax.ShapeDtypeStruct((BATCH, EMBED_DIM), jnp.float32),
        compiler_params=pltpu.CompilerParams(use_tc_tiling_on_sc=False),
        scratch_types=(
            [pltpu.VMEM((NCHUNK, CHUNK_IDX), jnp.int32),
             pltpu.VMEM((BPW, EMBED_DIM), jnp.float32)]
            + [pltpu.VMEM((CHUNK_IDX, EMBED_DIM), jnp.float32)
               for _ in range(NBUF)]
            + [pltpu.SemaphoreType.DMA for _ in range(NBUF)]
        ),
    )
    def emb_kernel(table_hbm, idx_hbm, out_hbm, idx_v, out_v, *bufs_sems):
        bufs = bufs_sems[:NBUF]
        sems = bufs_sems[NBUF:]
        wid = lax.axis_index("s") * NC + lax.axis_index("c")

        pltpu.sync_copy(idx_hbm.at[wid], idx_v)

        def start(c, b):
            @pl.when(c < NCHUNK)
            def _():
                pltpu.async_copy(table_hbm.at[idx_v.at[c]], bufs[b], sems[b])

        def wait(b):
            pltpu.make_async_copy(
                table_hbm.at[pl.ds(0, CHUNK_IDX)], bufs[b], sems[b]
            ).wait()

        for b in range(NBUF):
            start(b, b)

        def body(g, carry):
            for b in range(NBUF):
                c = g * NBUF + b
                wait(b)
                buf = bufs[b]
                for j in range(CB):
                    r0 = j * N_FIELDS
                    acc0 = buf[r0, pl.ds(0, LANES)]
                    acc1 = buf[r0, pl.ds(LANES, LANES)]
                    for f in range(1, N_FIELDS):
                        acc0 = acc0 + buf[r0 + f, pl.ds(0, LANES)]
                        acc1 = acc1 + buf[r0 + f, pl.ds(LANES, LANES)]
                    out_r = c * CB + j
                    out_v[out_r, pl.ds(0, LANES)] = acc0
                    out_v[out_r, pl.ds(LANES, LANES)] = acc1
                start(c + NBUF, b)
            return carry

        lax.fori_loop(0, NCHUNK // NBUF, body, 0)

        pltpu.sync_copy(out_v, out_hbm.at[pl.ds(wid * BPW, BPW)])

    return emb_kernel


_gather_kernel = _make_gather_kernel()


@jax.jit
def kernel(x, tables):
    tabt = jnp.transpose(tables, (0, 2, 1))
    flat128 = _transpose_flat(tabt)
    flat = flat128.reshape(FLAT_ROWS * 4, EMBED_DIM)
    xi = x.astype(jnp.int32)
    f_base = (jnp.arange(N_FIELDS, dtype=jnp.int32) * VPAD)[None, :]
    rows = (
        f_base
        + (xi // 512) * 512
        + (xi % 128) * 4
        + (xi // 128) % 4
    )
    flat_idx = rows.reshape(NW, NCHUNK, CHUNK_IDX)
    return _gather_kernel(flat, flat_idx)


# speedup vs baseline: 6.6855x; 1.0918x over previous
import functools

import jax
import jax.numpy as jnp
from jax import lax
from jax.experimental import pallas as pl
from jax.experimental.pallas import tpu as pltpu
from jax.experimental.pallas import tpu_sc as plsc

N_FIELDS = 26
VOCAB = 100000
EMBED_DIM = 32
BATCH = 16384

NC, NS, LANES = 2, 16, 16
NW = NC * NS

VBLK = 25600
NVBLK = -(-VOCAB // VBLK)
VPAD = NVBLK * VBLK
ROWS128 = VBLK * EMBED_DIM // 128
FLAT_ROWS = N_FIELDS * NVBLK * ROWS128

BPW = BATCH // NW
CB = 4
NCHUNK = BPW // CB
CHUNK_IDX = CB * N_FIELDS
NBUF = 4


def _transpose_body(in_ref, out_ref):
    blk = in_ref[0]
    d_io = lax.broadcasted_iota(jnp.int32, (EMBED_DIM, 128), 0)
    c_io = lax.broadcasted_iota(jnp.int32, (EMBED_DIM, 128), 1)
    sel = [
        jnp.where(c_io == d_io + EMBED_DIM * p, 1.0, 0.0)
        for p in range(4)
    ]
    for q in range(VBLK // 512):
        acc = None
        for p in range(4):
            t = lax.dot_general(
                blk[:, 128 * (4 * q + p):128 * (4 * q + p + 1)],
                sel[p],
                (((0,), (0,)), ((), ())),
            )
            acc = t if acc is None else acc + t
        out_ref[q * 128:(q + 1) * 128, :] = acc


_transpose_flat = pl.pallas_call(
    _transpose_body,
    grid=(N_FIELDS, NVBLK),
    in_specs=[
        pl.BlockSpec((1, EMBED_DIM, VBLK), lambda f, v: (f, 0, v)),
    ],
    out_specs=pl.BlockSpec((ROWS128, 128), lambda f, v: (f * NVBLK + v, 0)),
    out_shape=jax.ShapeDtypeStruct((FLAT_ROWS, 128), jnp.float32),
)


def _make_gather_kernel():
    mesh = plsc.VectorSubcoreMesh(core_axis_name="c", subcore_axis_name="s")

    @functools.partial(
        pl.kernel,
        mesh=mesh,
        out_type=jax.ShapeDtypeStruct((BATCH, EMBED_DIM), jnp.float32),
        compiler_params=pltpu.CompilerParams(use_tc_tiling_on_sc=False),
        scratch_types=(
            [pltpu.VMEM((NCHUNK, CHUNK_IDX), jnp.int32),
             pltpu.VMEM((BPW, EMBED_DIM), jnp.float32)]
            + [pltpu.VMEM((CHUNK_IDX, EMBED_DIM), jnp.float32)
               for _ in range(NBUF)]
            + [pltpu.SemaphoreType.DMA for _ in range(NBUF)]
        ),
    )
    def emb_kernel(table_hbm, idx_hbm, out_hbm, idx_v, out_v, *bufs_sems):
        bufs = bufs_sems[:NBUF]
        sems = bufs_sems[NBUF:]
        wid = lax.axis_index("s") * NC + lax.axis_index("c")

        pltpu.sync_copy(idx_hbm.at[wid], idx_v)

        def start(c, b):
            @pl.when(c < NCHUNK)
            def _():
                pltpu.async_copy(table_hbm.at[idx_v.at[c]], bufs[b], sems[b])

        def wait(b):
            pltpu.make_async_copy(
                table_hbm.at[pl.ds(0, CHUNK_IDX)], bufs[b], sems[b]
            ).wait()

        for b in range(NBUF):
            start(b, b)

        def body(g, carry):
            for b in range(NBUF):
                c = g * NBUF + b
                wait(b)
                buf = bufs[b]
                for j in range(CB):
                    r0 = j * N_FIELDS
                    acc0 = buf[r0, pl.ds(0, LANES)]
                    acc1 = buf[r0, pl.ds(LANES, LANES)]
                    for f in range(1, N_FIELDS):
                        acc0 = acc0 + buf[r0 + f, pl.ds(0, LANES)]
                        acc1 = acc1 + buf[r0 + f, pl.ds(LANES, LANES)]
                    out_r = c * CB + j
                    out_v[out_r, pl.ds(0, LANES)] = acc0
                    out_v[out_r, pl.ds(LANES, LANES)] = acc1
                start(c + NBUF, b)
            return carry

        lax.fori_loop(0, NCHUNK // NBUF, body, 0)

        pltpu.sync_copy(out_v, out_hbm.at[pl.ds(wid * BPW, BPW)])

    return emb_kernel


_gather_kernel = _make_gather_kernel()


@jax.jit
def kernel(x, tables):
    tabt = jnp.transpose(tables, (0, 2, 1))
    flat128 = _transpose_flat(tabt)
    flat = flat128.reshape(FLAT_ROWS * 4, EMBED_DIM)
    xi = x.astype(jnp.int32)
    f_base = (jnp.arange(N_FIELDS, dtype=jnp.int32) * VPAD)[None, :]
    rows = (
        f_base
        + (xi // 512) * 512
        + (xi % 128) * 4
        + (xi // 128) % 4
    )
    flat_idx = rows.reshape(NW, NCHUNK, CHUNK_IDX)
    return _gather_kernel(flat, flat_idx)


# speedup vs baseline: 6.8141x; 1.0192x over previous
import functools

import jax
import jax.numpy as jnp
from jax import lax
from jax.experimental import pallas as pl
from jax.experimental.pallas import tpu as pltpu
from jax.experimental.pallas import tpu_sc as plsc

N_FIELDS = 26
VOCAB = 100000
EMBED_DIM = 32
BATCH = 16384

NC, NS, LANES = 2, 16, 16
NW = NC * NS

VBLK = 51200
NVBLK = -(-VOCAB // VBLK)
VPAD = NVBLK * VBLK
ROWS128 = VBLK * EMBED_DIM // 128
FLAT_ROWS = N_FIELDS * NVBLK * ROWS128

BPW = BATCH // NW
CB = 4
NCHUNK = BPW // CB
CHUNK_IDX = CB * N_FIELDS
NBUF = 4


def _transpose_body(in_ref, out_ref):
    blk = in_ref[0]
    d_io = lax.broadcasted_iota(jnp.int32, (EMBED_DIM, 128), 0)
    c_io = lax.broadcasted_iota(jnp.int32, (EMBED_DIM, 128), 1)
    sel = [
        jnp.where(c_io == d_io + EMBED_DIM * p, 1.0, 0.0)
        for p in range(4)
    ]
    for q in range(VBLK // 512):
        acc = None
        for p in range(4):
            t = lax.dot_general(
                blk[:, 128 * (4 * q + p):128 * (4 * q + p + 1)],
                sel[p],
                (((0,), (0,)), ((), ())),
            )
            acc = t if acc is None else acc + t
        out_ref[q * 128:(q + 1) * 128, :] = acc


_transpose_flat = pl.pallas_call(
    _transpose_body,
    grid=(N_FIELDS, NVBLK),
    in_specs=[
        pl.BlockSpec((1, EMBED_DIM, VBLK), lambda f, v: (f, 0, v)),
    ],
    out_specs=pl.BlockSpec((ROWS128, 128), lambda f, v: (f * NVBLK + v, 0)),
    out_shape=jax.ShapeDtypeStruct((FLAT_ROWS, 128), jnp.float32),
)


def _make_gather_kernel():
    mesh = plsc.VectorSubcoreMesh(core_axis_name="c", subcore_axis_name="s")

    @functools.partial(
        pl.kernel,
        mesh=mesh,
        out_type=jax.ShapeDtypeStruct((BATCH, EMBED_DIM), jnp.float32),
        compiler_params=pltpu.CompilerParams(use_tc_tiling_on_sc=False),
        scratch_types=(
            [pltpu.VMEM((NCHUNK, CHUNK_IDX), jnp.int32),
             pltpu.VMEM((BPW, EMBED_DIM), jnp.float32)]
            + [pltpu.VMEM((CHUNK_IDX, EMBED_DIM), jnp.float32)
               for _ in range(NBUF)]
            + [pltpu.SemaphoreType.DMA for _ in range(NBUF)]
        ),
    )
    def emb_kernel(table_hbm, idx_hbm, out_hbm, idx_v, out_v, *bufs_sems):
        bufs = bufs_sems[:NBUF]
        sems = bufs_sems[NBUF:]
        wid = lax.axis_index("s") * NC + lax.axis_index("c")

        pltpu.sync_copy(idx_hbm.at[wid], idx_v)

        def start(c, b):
            @pl.when(c < NCHUNK)
            def _():
                pltpu.async_copy(table_hbm.at[idx_v.at[c]], bufs[b], sems[b])

        def wait(b):
            pltpu.make_async_copy(
                table_hbm.at[pl.ds(0, CHUNK_IDX)], bufs[b], sems[b]
            ).wait()

        for b in range(NBUF):
            start(b, b)

        def body(g, carry):
            for b in range(NBUF):
                c = g * NBUF + b
                wait(b)
                buf = bufs[b]
                for j in range(CB):
                    r0 = j * N_FIELDS
                    acc0 = buf[r0, pl.ds(0, LANES)]
                    acc1 = buf[r0, pl.ds(LANES, LANES)]
                    for f in range(1, N_FIELDS):
                        acc0 = acc0 + buf[r0 + f, pl.ds(0, LANES)]
                        acc1 = acc1 + buf[r0 + f, pl.ds(LANES, LANES)]
                    out_r = c * CB + j
                    out_v[out_r, pl.ds(0, LANES)] = acc0
                    out_v[out_r, pl.ds(LANES, LANES)] = acc1
                start(c + NBUF, b)
            return carry

        lax.fori_loop(0, NCHUNK // NBUF, body, 0)

        pltpu.sync_copy(out_v, out_hbm.at[pl.ds(wid * BPW, BPW)])

    return emb_kernel


_gather_kernel = _make_gather_kernel()


@jax.jit
def kernel(x, tables):
    tabt = jnp.transpose(tables, (0, 2, 1))
    flat128 = _transpose_flat(tabt)
    flat = flat128.reshape(FLAT_ROWS * 4, EMBED_DIM)
    xi = x.astype(jnp.int32)
    f_base = (jnp.arange(N_FIELDS, dtype=jnp.int32) * VPAD)[None, :]
    rows = (
        f_base
        + (xi // 512) * 512
        + (xi % 128) * 4
        + (xi // 128) % 4
    )
    flat_idx = rows.reshape(NW, NCHUNK, CHUNK_IDX)
    return _gather_kernel(flat, flat_idx)
